# baseline (speedup 1.0000x reference)
import jax
import jax.numpy as jnp
from jax.experimental import pallas as pl
from jax.experimental.pallas import tpu as pltpu
from jax.experimental.pallas import tpu_sc as plsc

_BN = 512
_NB = 16
_GW = 128


def _tc_body(th_ref, wt_ref, b_ref, wv_ref, m_ref, w_ref, vext_ref, gidx_ref):
    i = pl.program_id(0)
    bn, C = w_ref.shape
    N = th_ref.shape[1]
    th = th_ref[:, pl.ds(i * bn, bn)]
    w = jax.lax.dot_general(th, wt_ref[...], (((0,), (1,)), ((), ())),
                            preferred_element_type=jnp.float32)
    w = w + b_ref[...]
    w_ref[...] = w
    rowmax = jnp.max(w, axis=1, keepdims=True)
    rowio_f = jax.lax.broadcasted_iota(
        jnp.int32, (bn, 1), 0).astype(jnp.float32)
    cand = jnp.where(w == rowmax, rowio_f, -1.0)
    blk_f = jnp.max(cand, axis=0, keepdims=True)
    tiesum = jnp.sum(cand)
    expected = float(bn * (bn - 1) // 2 - bn * (C - 1))
    noties = tiesum == expected

    def _update(blkmax):
        @pl.when(i == 0)
        def _():
            gidx_ref[...] = blkmax

        @pl.when(i > 0)
        def _():
            gidx_ref[...] = jnp.maximum(gidx_ref[...], blkmax)

    @pl.when(noties)
    def _fast():
        _update(jnp.where(blk_f >= 0.0,
                          blk_f.astype(jnp.int32) + i * bn, -1))

    @pl.when(jnp.logical_not(noties))
    def _exact():
        w2 = w_ref[...]
        rowmax2 = jnp.max(w2, axis=1, keepdims=True)
        colio = jax.lax.broadcasted_iota(jnp.int32, (bn, C), 1)
        idx = jnp.min(jnp.where(w2 == rowmax2, colio, C), axis=1,
                      keepdims=True)
        rowio = jax.lax.broadcasted_iota(jnp.int32, (bn, C), 0) + i * bn
        _update(jnp.max(jnp.where(colio == idx, rowio, -1), axis=0,
                        keepdims=True))

    @pl.when(i == _NB - 1)
    def _finish():
        vext_ref[pl.ds(0, N), :] = jax.lax.dot_general(
            th_ref[...], wv_ref[...], (((0,), (0,)), ((), ())),
            preferred_element_type=jnp.float32)
        vext_ref[pl.ds(N, C), :] = m_ref[...]
        wr = gidx_ref[...]
        cio = jax.lax.broadcasted_iota(jnp.int32, wr.shape, 1)
        gidx_ref[...] = jnp.where(wr >= 0, wr, N + cio)


def _tc_call(thetas, W, b2, Wv, M):
    IN, N = thetas.shape
    C = W.shape[0]
    L = Wv.shape[1]
    grid = (_NB,)
    return pl.pallas_call(
        _tc_body,
        grid=grid,
        in_specs=[
            pl.BlockSpec((IN, N), lambda i: (0, 0)),
            pl.BlockSpec((C, IN), lambda i: (0, 0)),
            pl.BlockSpec((1, C), lambda i: (0, 0)),
            pl.BlockSpec((IN, L), lambda i: (0, 0)),
            pl.BlockSpec((C, L), lambda i: (0, 0)),
        ],
        out_specs=[
            pl.BlockSpec((_BN, C), lambda i: (i, 0)),
            pl.BlockSpec((N + C, L), lambda i: (0, 0)),
            pl.BlockSpec((1, C), lambda i: (0, 0)),
        ],
        out_shape=[
            jax.ShapeDtypeStruct((N, C), jnp.float32),
            jax.ShapeDtypeStruct((N + C, L), jnp.float32),
            jax.ShapeDtypeStruct((1, C), jnp.int32),
        ],
    )(thetas, W, b2, Wv, M)


def _sc_gather(src, gidx):
    C = gidx.shape[1]
    L = src.shape[1]
    mesh = plsc.VectorSubcoreMesh(core_axis_name="c", subcore_axis_name="s")

    @pl.kernel(out_type=jax.ShapeDtypeStruct((C, L), src.dtype), mesh=mesh)
    def k(src_hbm, i_hbm, o_hbm):
        def body(i_vmem, o_vmem):
            pltpu.sync_copy(src_hbm.at[i_vmem.at[0]], o_vmem)

        pltpu.emit_pipeline(
            body,
            grid=(C // _GW,),
            in_specs=[pl.BlockSpec((1, _GW), index_map=lambda i: (0, i))],
            out_specs=[pl.BlockSpec((_GW, L), index_map=lambda i: (i, 0))],
            core_axis_name=("c", "s"),
            dimension_semantics=(pltpu.PARALLEL,),
        )(i_hbm, o_hbm)

    return k(src, gidx)


def kernel(thetas, W, b, M, Wv):
    C, L = M.shape
    w, vext, gidx = _tc_call(thetas, W, b.reshape(1, C), Wv, M)
    M_new = _sc_gather(vext, gidx)
    return (w, M_new)


# speedup vs baseline: 1.0618x; 1.0618x over previous
import jax
import jax.numpy as jnp
from jax.experimental import pallas as pl
from jax.experimental.pallas import tpu as pltpu
from jax.experimental.pallas import tpu_sc as plsc

_BN = 512
_NB = 16
_GW = 128


def _tc_body(th_ref, wt_ref, b_ref, wv_ref, m_ref, w_ref, vext_ref, gidx_ref):
    i = pl.program_id(0)
    bn, C = w_ref.shape
    N = th_ref.shape[1]
    th = th_ref[:, pl.ds(i * bn, bn)]
    w = jax.lax.dot_general(th, wt_ref[...], (((0,), (0,)), ((), ())),
                            preferred_element_type=jnp.float32)
    w = w + b_ref[...]
    w_ref[...] = w
    rowmax = jnp.max(w, axis=1, keepdims=True)
    rowio_f = jax.lax.broadcasted_iota(
        jnp.int32, (bn, 1), 0).astype(jnp.float32)
    cand = jnp.where(w == rowmax, rowio_f, -1.0)
    blk_f = jnp.max(cand, axis=0, keepdims=True)
    tiesum = jnp.sum(cand)
    expected = float(bn * (bn - 1) // 2 - bn * (C - 1))
    noties = tiesum == expected

    def _update(blkmax):
        @pl.when(i == 0)
        def _():
            gidx_ref[...] = blkmax

        @pl.when(i > 0)
        def _():
            gidx_ref[...] = jnp.maximum(gidx_ref[...], blkmax)

    @pl.when(noties)
    def _fast():
        _update(jnp.where(blk_f >= 0.0,
                          blk_f.astype(jnp.int32) + i * bn, -1))

    @pl.when(jnp.logical_not(noties))
    def _exact():
        w2 = w_ref[...]
        rowmax2 = jnp.max(w2, axis=1, keepdims=True)
        colio = jax.lax.broadcasted_iota(jnp.int32, (bn, C), 1)
        idx = jnp.min(jnp.where(w2 == rowmax2, colio, C), axis=1,
                      keepdims=True)
        rowio = jax.lax.broadcasted_iota(jnp.int32, (bn, C), 0) + i * bn
        _update(jnp.max(jnp.where(colio == idx, rowio, -1), axis=0,
                        keepdims=True))

    @pl.when(i == _NB - 1)
    def _finish():
        vext_ref[pl.ds(0, N), :] = jax.lax.dot_general(
            th_ref[...], wv_ref[...], (((0,), (0,)), ((), ())),
            preferred_element_type=jnp.float32)
        vext_ref[pl.ds(N, C), :] = m_ref[...]
        wr = gidx_ref[...]
        cio = jax.lax.broadcasted_iota(jnp.int32, wr.shape, 1)
        gidx_ref[...] = jnp.where(wr >= 0, wr, N + cio)


def _tc_call(thetas, Wt, b2, Wv, M):
    IN, N = thetas.shape
    C = Wt.shape[1]
    L = Wv.shape[1]
    grid = (_NB,)
    return pl.pallas_call(
        _tc_body,
        grid=grid,
        in_specs=[
            pl.BlockSpec((IN, N), lambda i: (0, 0)),
            pl.BlockSpec((IN, C), lambda i: (0, 0)),
            pl.BlockSpec((1, C), lambda i: (0, 0)),
            pl.BlockSpec((IN, L), lambda i: (0, 0)),
            pl.BlockSpec((C, L), lambda i: (0, 0)),
        ],
        out_specs=[
            pl.BlockSpec((_BN, C), lambda i: (i, 0)),
            pl.BlockSpec((N + C, L), lambda i: (0, 0)),
            pl.BlockSpec((1, C), lambda i: (0, 0)),
        ],
        out_shape=[
            jax.ShapeDtypeStruct((N, C), jnp.float32),
            jax.ShapeDtypeStruct((N + C, L), jnp.float32),
            jax.ShapeDtypeStruct((1, C), jnp.int32),
        ],
    )(thetas, Wt, b2, Wv, M)


def _sc_gather(src, gidx):
    C = gidx.shape[1]
    L = src.shape[1]
    mesh = plsc.VectorSubcoreMesh(core_axis_name="c", subcore_axis_name="s")

    @pl.kernel(out_type=jax.ShapeDtypeStruct((C, L), src.dtype), mesh=mesh)
    def k(src_hbm, i_hbm, o_hbm):
        def body(i_vmem, o_vmem):
            pltpu.sync_copy(src_hbm.at[i_vmem.at[0]], o_vmem)

        pltpu.emit_pipeline(
            body,
            grid=(C // _GW,),
            in_specs=[pl.BlockSpec((1, _GW), index_map=lambda i: (0, i))],
            out_specs=[pl.BlockSpec((_GW, L), index_map=lambda i: (i, 0))],
            core_axis_name=("c", "s"),
            dimension_semantics=(pltpu.PARALLEL,),
        )(i_hbm, o_hbm)

    return k(src, gidx)


def kernel(thetas, W, b, M, Wv):
    C, L = M.shape
    w, vext, gidx = _tc_call(thetas, W.T, b.reshape(1, C), Wv, M)
    M_new = _sc_gather(vext, gidx)
    return (w, M_new)
